# baseline (speedup 1.0000x reference)
import jax
import jax.numpy as jnp
from jax import lax
from jax.experimental import pallas as pl
from jax.experimental.pallas import tpu as pltpu
from jax.experimental.pallas import tpu_sc as plsc

_L = 3
_NC, _NS, _LANES = 2, 16, 16


def _sc_adj_body(ei_hbm, ea_hbm, a_hbm,
                 ei_v, cidx_v, ea_v, zfill_v, a_sh,
                 sem_z, sem_e, sem_i, sem_a):
    c = lax.axis_index("c")
    s = lax.axis_index("s")
    nsq = a_sh.shape[0] // 2
    epw = ea_v.shape[0]
    nchunk = epw // 128
    zrows = a_sh.shape[0] // _NS
    ebase = c * (_NS * epw) + s * epw

    ecp = pltpu.async_copy(ea_hbm.at[pl.ds(ebase, epw)], ea_v, sem_e)
    g = c * 2 + s // 8
    k = s % 8
    icp = pltpu.async_copy(ei_hbm.at[g], ei_v, sem_i)

    zr = zfill_v.shape[0]

    def _zf(i, carry):
        for j in range(8):
            zfill_v[i, pl.ds(j * _LANES, _LANES)] = jnp.zeros(
                (_LANES,), jnp.float32)
        return carry
    lax.fori_loop(0, zr, _zf, 0)
    zcps = [pltpu.async_copy(
        zfill_v, a_sh.at[pl.ds(s * zrows + k * zr, zr)], sem_z)
        for k in range(zrows // zr)]

    icp.wait()

    off = (s // 8) * nsq
    off_v = jnp.full((_LANES,), off, jnp.int32)
    for ch in range(nchunk):
        for i in range(128 // _LANES):
            sl = pl.ds(i * _LANES, _LANES)
            esl = pl.ds(k * epw + ch * 128 + i * _LANES, _LANES)
            cidx_v[ch, sl] = (ei_v[0, esl] * 64
                              + ei_v[1, esl] + off_v)

    ecp.wait()
    for cp in zcps:
        cp.wait()
    plsc.subcore_barrier()

    acps = [pltpu.async_copy(ea_v.at[pl.ds(ch * 128, 128)],
                             a_sh.at[cidx_v.at[ch]], sem_a, add=True)
            for ch in range(nchunk)]
    for cp in acps:
        cp.wait()
    plsc.subcore_barrier()

    base = c * a_sh.shape[0] + s * zrows
    pltpu.sync_copy(a_sh.at[pl.ds(s * zrows, zrows)],
                    a_hbm.at[pl.ds(base, zrows)])


def _make_sc_adj(bsz, n, d, be):
    mesh = plsc.VectorSubcoreMesh(core_axis_name="c", subcore_axis_name="s")
    epw = be // (_NC * _NS)
    e = be // bsz
    sc_rows = (bsz // _NC) * n * n
    return pl.kernel(
        _sc_adj_body,
        out_type=jax.ShapeDtypeStruct((bsz * n * n, d), jnp.float32),
        mesh=mesh,
        scratch_types=[
            pltpu.VMEM((2, e), jnp.int32),
            pltpu.VMEM((epw // 128, 128), jnp.int32),
            pltpu.VMEM((epw, d), jnp.float32),
            pltpu.VMEM((128, d), jnp.float32),
            pltpu.VMEM_SHARED((sc_rows, d), jnp.float32),
            pltpu.SemaphoreType.DMA,
            pltpu.SemaphoreType.DMA,
            pltpu.SemaphoreType.DMA,
            pltpu.SemaphoreType.DMA,
        ],
    )


def _tc_body(x_ref, a_ref, w_ref, b_ref, g_ref, be_ref, ge_ref, out_ref):
    x0 = x_ref[0]
    n = x0.shape[0]
    x = x0
    for i in range(_L):
        residual = x
        h = jnp.dot(x, w_ref[i], preferred_element_type=jnp.float32)
        out = jnp.zeros_like(h)
        for m in range(n):
            out = out + a_ref[0, m] * h[m:m + 1, :]
        out = out + b_ref[i]
        x = jnp.maximum(out, 0.0)
        mu = jnp.mean(x, axis=-1, keepdims=True)
        var = jnp.mean((x - mu) * (x - mu), axis=-1, keepdims=True)
        x = (x - mu) * lax.rsqrt(var + 1e-5) * g_ref[i] + be_ref[i]
        if i > 0:
            x = x + residual

    presence = (jnp.sum(x0, axis=1, keepdims=True) != 0.0
                ).astype(jnp.float32)
    out_ref[0] = x * presence + ge_ref[...] * (1.0 - presence)


def kernel(node_features, edge_indices, edge_attrs, W, b, gamma, beta,
           global_emb):
    bsz, n, d = node_features.shape
    e = edge_attrs.shape[1]
    be = bsz * e

    ei = edge_indices.astype(jnp.int32)
    ea = edge_attrs.reshape(be, d)

    a_flat = _make_sc_adj(bsz, n, d, be)(ei, ea)
    a = a_flat.reshape(bsz, n, n, d)

    grid = (bsz,)
    out = pl.pallas_call(
        _tc_body,
        grid=grid,
        in_specs=[
            pl.BlockSpec((1, n, d), lambda g: (g, 0, 0)),
            pl.BlockSpec((1, n, n, d), lambda g: (g, 0, 0, 0)),
            pl.BlockSpec((_L, d, d), lambda g: (0, 0, 0)),
            pl.BlockSpec((_L, d), lambda g: (0, 0)),
            pl.BlockSpec((_L, d), lambda g: (0, 0)),
            pl.BlockSpec((_L, d), lambda g: (0, 0)),
            pl.BlockSpec((n, d), lambda g: (0, 0)),
        ],
        out_specs=pl.BlockSpec((1, n, d), lambda g: (g, 0, 0)),
        out_shape=jax.ShapeDtypeStruct((bsz, n, d), jnp.float32),
    )(node_features, a, W, b, gamma, beta, global_emb)
    return out


# speedup vs baseline: 1.0260x; 1.0260x over previous
import jax
import jax.numpy as jnp
from jax import lax
from jax.experimental import pallas as pl
from jax.experimental.pallas import tpu as pltpu
from jax.experimental.pallas import tpu_sc as plsc

_L = 3
_NC, _NS, _LANES = 2, 16, 16


def _sc_adj_body(ei_hbm, ea_hbm, a_hbm,
                 ei_v, cidx_v, ea_v, zfill_v, a_sh,
                 sem_z, sem_e, sem_i, sem_a):
    c = lax.axis_index("c")
    s = lax.axis_index("s")
    nsq = a_sh.shape[0] // 2
    epw = ea_v.shape[0]
    nchunk = epw // 128
    zrows = a_sh.shape[0] // _NS
    ebase = c * (_NS * epw) + s * epw

    ecp = pltpu.async_copy(ea_hbm.at[pl.ds(ebase, epw)], ea_v, sem_e)
    g = c * 2 + s // 8
    k = s % 8
    icp = pltpu.async_copy(ei_hbm.at[g], ei_v, sem_i)

    zr = zfill_v.shape[0]

    def _zf(i, carry):
        for j in range(8):
            zfill_v[i, pl.ds(j * _LANES, _LANES)] = jnp.zeros(
                (_LANES,), jnp.float32)
        return carry
    lax.fori_loop(0, zr, _zf, 0)
    zcps = [pltpu.async_copy(
        zfill_v, a_sh.at[pl.ds(s * zrows + k * zr, zr)], sem_z)
        for k in range(zrows // zr)]

    icp.wait()

    off = (s // 8) * nsq
    off_v = jnp.full((_LANES,), off, jnp.int32)
    for ch in range(nchunk):
        for i in range(128 // _LANES):
            sl = pl.ds(i * _LANES, _LANES)
            esl = pl.ds(k * epw + ch * 128 + i * _LANES, _LANES)
            cidx_v[ch, sl] = (ei_v[0, esl] * 64
                              + ei_v[1, esl] + off_v)

    ecp.wait()
    for cp in zcps:
        cp.wait()
    plsc.subcore_barrier()

    acps = [pltpu.async_copy(ea_v.at[pl.ds(ch * 128, 128)],
                             a_sh.at[cidx_v.at[ch]], sem_a, add=True)
            for ch in range(nchunk)]
    for cp in acps:
        cp.wait()
    plsc.subcore_barrier()

    base = c * a_sh.shape[0] + s * zrows
    pltpu.sync_copy(a_sh.at[pl.ds(s * zrows, zrows)],
                    a_hbm.at[pl.ds(base, zrows)])


def _make_sc_adj(bsz, n, d, be):
    mesh = plsc.VectorSubcoreMesh(core_axis_name="c", subcore_axis_name="s")
    epw = be // (_NC * _NS)
    e = be // bsz
    sc_rows = (bsz // _NC) * n * n
    return pl.kernel(
        _sc_adj_body,
        out_type=jax.ShapeDtypeStruct((bsz * n * n, d), jnp.float32),
        mesh=mesh,
        scratch_types=[
            pltpu.VMEM((2, e), jnp.int32),
            pltpu.VMEM((epw // 128, 128), jnp.int32),
            pltpu.VMEM((epw, d), jnp.float32),
            pltpu.VMEM((128, d), jnp.float32),
            pltpu.VMEM_SHARED((sc_rows, d), jnp.float32),
            pltpu.SemaphoreType.DMA,
            pltpu.SemaphoreType.DMA,
            pltpu.SemaphoreType.DMA,
            pltpu.SemaphoreType.DMA,
        ],
    )


def _tc_body(x_ref, a_ref, w_ref, b_ref, g_ref, be_ref, ge_ref, out_ref):
    bsz = x_ref.shape[0]
    n = x_ref.shape[1]
    for gi in range(bsz):
        x0 = x_ref[gi]
        x = x0
        for i in range(_L):
            residual = x
            h = jnp.dot(x, w_ref[i], preferred_element_type=jnp.float32)
            out = jnp.zeros_like(h)
            for m in range(n):
                out = out + a_ref[gi, m] * h[m:m + 1, :]
            out = out + b_ref[i]
            x = jnp.maximum(out, 0.0)
            mu = jnp.mean(x, axis=-1, keepdims=True)
            var = jnp.mean((x - mu) * (x - mu), axis=-1, keepdims=True)
            x = (x - mu) * lax.rsqrt(var + 1e-5) * g_ref[i] + be_ref[i]
            if i > 0:
                x = x + residual

        presence = (jnp.sum(x0, axis=1, keepdims=True) != 0.0
                    ).astype(jnp.float32)
        out_ref[gi] = x * presence + ge_ref[...] * (1.0 - presence)


def kernel(node_features, edge_indices, edge_attrs, W, b, gamma, beta,
           global_emb):
    bsz, n, d = node_features.shape
    e = edge_attrs.shape[1]
    be = bsz * e

    ei = edge_indices.astype(jnp.int32)
    ea = edge_attrs.reshape(be, d)

    a_flat = _make_sc_adj(bsz, n, d, be)(ei, ea)
    a = a_flat.reshape(bsz, n, n, d)

    out = pl.pallas_call(
        _tc_body,
        out_shape=jax.ShapeDtypeStruct((bsz, n, d), jnp.float32),
    )(node_features, a, W, b, gamma, beta, global_emb)
    return out


# speedup vs baseline: 1.0281x; 1.0020x over previous
import jax
import jax.numpy as jnp
from jax import lax
from jax.experimental import pallas as pl
from jax.experimental.pallas import tpu as pltpu
from jax.experimental.pallas import tpu_sc as plsc

_L = 3
_NC, _NS, _LANES = 2, 16, 16


def _sc_adj_body(ei_hbm, ea_hbm, a_hbm,
                 ei_v, cidx_v, ea_v, zfill_v, a_sh,
                 sem_z, sem_e, sem_i, sem_a):
    c = lax.axis_index("c")
    s = lax.axis_index("s")
    nsq = a_sh.shape[0] // 2
    epw = ea_v.shape[0]
    nchunk = epw // 128
    zrows = a_sh.shape[0] // _NS
    ebase = c * (_NS * epw) + s * epw

    ecp = pltpu.async_copy(ea_hbm.at[pl.ds(ebase, epw)], ea_v, sem_e)
    g = c * 2 + s // 8
    k = s % 8
    icp = pltpu.async_copy(ei_hbm.at[g], ei_v, sem_i)

    zr = zfill_v.shape[0]

    def _zf(i, carry):
        for j in range(8):
            zfill_v[i, pl.ds(j * _LANES, _LANES)] = jnp.zeros(
                (_LANES,), jnp.float32)
        return carry
    lax.fori_loop(0, zr, _zf, 0)
    zcps = [pltpu.async_copy(
        zfill_v, a_sh.at[pl.ds(s * zrows + kz * zr, zr)], sem_z)
        for kz in range(zrows // zr)]

    icp.wait()

    n_nodes = int(round(nsq ** 0.5))
    off = (s // 8) * nsq
    off_v = jnp.full((_LANES,), off, jnp.int32)
    for ch in range(nchunk):
        for i in range(128 // _LANES):
            sl = pl.ds(i * _LANES, _LANES)
            esl = pl.ds(k * epw + ch * 128 + i * _LANES, _LANES)
            cidx_v[ch, sl] = (ei_v[0, esl] * n_nodes
                              + ei_v[1, esl] + off_v)

    ecp.wait()
    for cp in zcps:
        cp.wait()
    plsc.subcore_barrier()

    acps = [pltpu.async_copy(ea_v.at[pl.ds(ch * 128, 128)],
                             a_sh.at[cidx_v.at[ch]], sem_a, add=True)
            for ch in range(nchunk)]
    for cp in acps:
        cp.wait()
    plsc.subcore_barrier()

    base = c * a_sh.shape[0] + s * zrows
    pltpu.sync_copy(a_sh.at[pl.ds(s * zrows, zrows)],
                    a_hbm.at[pl.ds(base, zrows)])


def _make_sc_adj(bsz, n, d, be):
    mesh = plsc.VectorSubcoreMesh(core_axis_name="c", subcore_axis_name="s")
    epw = be // (_NC * _NS)
    e = be // bsz
    sc_rows = (bsz // _NC) * n * n
    return pl.kernel(
        _sc_adj_body,
        out_type=jax.ShapeDtypeStruct((bsz * n * n, d), jnp.float32),
        mesh=mesh,
        scratch_types=[
            pltpu.VMEM((2, e), jnp.int32),
            pltpu.VMEM((epw // 128, 128), jnp.int32),
            pltpu.VMEM((epw, d), jnp.float32),
            pltpu.VMEM((128, d), jnp.float32),
            pltpu.VMEM_SHARED((sc_rows, d), jnp.float32),
            pltpu.SemaphoreType.DMA,
            pltpu.SemaphoreType.DMA,
            pltpu.SemaphoreType.DMA,
            pltpu.SemaphoreType.DMA,
        ],
    )


def _tc_body(x_ref, a_ref, w_ref, b_ref, g_ref, be_ref, ge_ref, out_ref):
    bsz = x_ref.shape[0]
    n = x_ref.shape[1]
    for gi in range(bsz):
        x0 = x_ref[gi]
        x = x0
        for i in range(_L):
            residual = x
            h = jnp.dot(x, w_ref[i], preferred_element_type=jnp.float32)
            out = jnp.zeros_like(h)
            for m in range(n):
                out = out + a_ref[gi, m] * h[m:m + 1, :]
            out = out + b_ref[i]
            x = jnp.maximum(out, 0.0)
            mu = jnp.mean(x, axis=-1, keepdims=True)
            var = jnp.mean((x - mu) * (x - mu), axis=-1, keepdims=True)
            x = (x - mu) * lax.rsqrt(var + 1e-5) * g_ref[i] + be_ref[i]
            if i > 0:
                x = x + residual

        presence = (jnp.sum(x0, axis=1, keepdims=True) != 0.0
                    ).astype(jnp.float32)
        out_ref[gi] = x * presence + ge_ref[...] * (1.0 - presence)


def kernel(node_features, edge_indices, edge_attrs, W, b, gamma, beta,
           global_emb):
    bsz, n, d = node_features.shape
    e = edge_attrs.shape[1]
    be = bsz * e

    ei = edge_indices.astype(jnp.int32)
    ea = edge_attrs.reshape(be, d)

    a_flat = _make_sc_adj(bsz, n, d, be)(ei, ea)
    a = a_flat.reshape(bsz, n, n, d)

    out = pl.pallas_call(
        _tc_body,
        out_shape=jax.ShapeDtypeStruct((bsz, n, d), jnp.float32),
    )(node_features, a, W, b, gamma, beta, global_emb)
    return out
